# baseline (speedup 1.0000x reference)
import functools

import numpy as np
import jax
import jax.numpy as jnp
from jax import lax
from jax.experimental import pallas as pl
from jax.experimental.pallas import tpu as pltpu
from jax.experimental.pallas import tpu_sc as plsc

N_REGIONS = 100000
N_EMB = 16
N_INTER = 16
N_CELLS = 256
N_REG_B = 4096

NC = 2
NS = 16
NW = NC * NS
BPW = N_REG_B // NW
GPW = BPW // 16
N_GROUPS = N_REG_B // 16


@functools.cache
def _make_sc_gather():
  mesh = plsc.VectorSubcoreMesh(core_axis_name="c", subcore_axis_name="s")

  @functools.partial(
    pl.kernel,
    mesh=mesh,
    out_type=(
        jax.ShapeDtypeStruct((N_GROUPS * 256, 16), jnp.float32),
        jax.ShapeDtypeStruct((N_REG_B, N_INTER), jnp.float32),
        jax.ShapeDtypeStruct((N_REG_B, N_INTER), jnp.float32),
        jax.ShapeDtypeStruct((N_REG_B, 16), jnp.float32),
    ),
    scratch_types=(
        pltpu.VMEM((BPW,), jnp.int32),
        pltpu.VMEM((BPW,), jnp.int32),
        pltpu.VMEM((16, 128), jnp.int32),
        pltpu.VMEM((GPW * 256, 16), jnp.float32),
        pltpu.VMEM((BPW, N_INTER), jnp.float32),
        pltpu.VMEM((BPW, N_INTER), jnp.float32),
        pltpu.VMEM((BPW, 16), jnp.float32),
        pltpu.SemaphoreType.DMA,
    ),
    compiler_params=pltpu.CompilerParams(use_tc_tiling_on_sc=False),
  )
  def _sc_gather(w1_hbm, b1_hbm, w2_hbm, b2_hbm, ix_hbm,
                 w1o, b1o, w2o, b2o,
                 idx_v, idxhi_v, idx2_v, w1cv, b1v, w2v, b2v, sem):
    wid = lax.axis_index("s") * NC + lax.axis_index("c")
    base = wid * BPW
    pltpu.sync_copy(ix_hbm.at[pl.ds(base, BPW)], idx_v)

    for j in range(BPW // 16):
        v = idx_v[pl.ds(j * 16, 16)]
        idxhi_v[pl.ds(j * 16, 16)] = lax.shift_right_logical(v, 4)

    for m in range(GPW * 16):
        g, c = m // 16, m % 16
        v = idx_v[pl.ds(g * 16, 16)]
        idx2_v[m // 8, pl.ds((m % 8) * 16, 16)] = v * 16 + c

    for j in range(16):
        pltpu.async_copy(w1_hbm.at[idx2_v.at[j]],
                         w1cv.at[pl.ds(j * 128, 128)], sem).wait()
    pltpu.async_copy(b1_hbm.at[idx_v], b1v, sem).wait()
    pltpu.async_copy(w2_hbm.at[idx_v], w2v, sem).wait()
    pltpu.async_copy(b2_hbm.at[idxhi_v], b2v, sem).wait()

    pltpu.sync_copy(w1cv, w1o.at[pl.ds(wid * GPW * 256, GPW * 256)])
    pltpu.sync_copy(b1v, b1o.at[pl.ds(base, BPW)])
    pltpu.sync_copy(w2v, w2o.at[pl.ds(base, BPW)])
    pltpu.sync_copy(b2v, b2o.at[pl.ds(base, BPW)])

  return _sc_gather


G_STEP = 16

_BLOCK_MASK = np.kron(np.eye(16, dtype=np.float32), np.ones((16, 16), np.float32))
_ONES_BD = np.kron(np.eye(G_STEP, dtype=np.float32),
                   np.kron(np.eye(16, dtype=np.float32), np.ones((16, 1), np.float32)))


def _tc_body(x3_ref, w1c_ref, b1_ref, w2_ref, b2t_ref, lo_ref, mask_ref,
             ones_ref, out_ref, t_ref):
    for g in range(G_STEP):
        xg = x3_ref[:, g, :].astype(jnp.bfloat16)
        wg = w1c_ref[g].astype(jnp.bfloat16)
        wt = jnp.concatenate([wg] * 16, axis=0)
        wbd = wt * mask_ref[...]
        h = jax.lax.dot_general(xg, wbd, (((1,), (0,)), ((), ())),
                                preferred_element_type=jnp.float32)
        h = h + b1_ref[g:g + 1, :]
        hs = 0.5 * jnp.tanh(0.5 * h) + 0.5
        t = hs * w2_ref[g:g + 1, :]
        t_ref[:, pl.ds(g * 256, 256)] = t.astype(jnp.bfloat16)

    out = jax.lax.dot_general(t_ref[...], ones_ref[...], (((1,), (0,)), ((), ())),
                              preferred_element_type=jnp.float32)

    lo = lo_ref[0:1, :]
    b2row = jnp.zeros((1, G_STEP * 16), jnp.float32)
    for l in range(16):
        b2row = b2row + jnp.where(lo == l, b2t_ref[l:l + 1, :], 0.0)

    out_ref[...] = out + b2row


def _tc_compute(x3, w1c3, b1m, w2m, b2t, lorow, maskc, onesbd):
    nb = N_GROUPS // G_STEP
    return pl.pallas_call(
        _tc_body,
        grid=(nb,),
        in_specs=[
            pl.BlockSpec((N_CELLS, G_STEP, 256), lambda i: (0, i, 0)),
            pl.BlockSpec((G_STEP, 16, 256), lambda i: (i, 0, 0)),
            pl.BlockSpec((G_STEP, 256), lambda i: (i, 0)),
            pl.BlockSpec((G_STEP, 256), lambda i: (i, 0)),
            pl.BlockSpec((16, G_STEP * 16), lambda i: (0, i)),
            pl.BlockSpec((1, G_STEP * 16), lambda i: (0, i)),
            pl.BlockSpec((256, 256), lambda i: (0, 0)),
            pl.BlockSpec((G_STEP * 256, G_STEP * 16), lambda i: (0, 0)),
        ],
        out_specs=pl.BlockSpec((N_CELLS, G_STEP * 16), lambda i: (0, i)),
        out_shape=jax.ShapeDtypeStruct((N_CELLS, N_REG_B), jnp.float32),
        scratch_shapes=[pltpu.VMEM((N_CELLS, G_STEP * 256), jnp.bfloat16)],
    )(x3, w1c3, b1m, w2m, b2t, lorow, maskc, onesbd)


def kernel(cell_region_embedding, region_ix, W1, b1, W2, b2):
    ix = region_ix.astype(jnp.int32)
    w1rows = W1.reshape(N_REGIONS * N_EMB, N_INTER)
    w2r = W2.reshape(N_REGIONS, N_INTER)
    b2r = b2.reshape(N_REGIONS // 16, 16)

    w1c, b1g, w2g, b2g = _make_sc_gather()(w1rows, b1, w2r, b2r, ix)

    x3 = cell_region_embedding.reshape(N_CELLS, N_GROUPS, 256)
    w1c3 = w1c.reshape(N_GROUPS, 16, 256)
    b1m = b1g.reshape(N_GROUPS, 256)
    w2m = w2g.reshape(N_GROUPS, 256)
    b2t = b2g.T
    lorow = (ix & 15).reshape(1, N_REG_B)
    maskc = jnp.asarray(_BLOCK_MASK, dtype=jnp.bfloat16)
    onesbd = jnp.asarray(_ONES_BD, dtype=jnp.bfloat16)

    return _tc_compute(x3, w1c3, b1m, w2m, b2t, lorow, maskc, onesbd)


# speedup vs baseline: 1.3618x; 1.3618x over previous
import functools

import jax
import jax.numpy as jnp
from jax import lax
from jax.experimental import pallas as pl
from jax.experimental.pallas import tpu as pltpu
from jax.experimental.pallas import tpu_sc as plsc

N_REGIONS = 100000
N_EMB = 16
N_INTER = 16
N_CELLS = 256
N_REG_B = 4096

NC = 2
NS = 16
NW = NC * NS
BPW = N_REG_B // NW


@functools.cache
def _make_sc_gather():
  mesh = plsc.VectorSubcoreMesh(core_axis_name="c", subcore_axis_name="s")

  @functools.partial(
    pl.kernel,
    mesh=mesh,
    out_type=(
        jax.ShapeDtypeStruct((N_EMB * N_INTER, N_REG_B), jnp.float32),
        jax.ShapeDtypeStruct((N_INTER, N_REG_B), jnp.float32),
        jax.ShapeDtypeStruct((N_INTER, N_REG_B), jnp.float32),
        jax.ShapeDtypeStruct((N_REG_B,), jnp.float32),
    ),
    scratch_types=(
        pltpu.VMEM((BPW,), jnp.int32),
        pltpu.VMEM((16, BPW), jnp.int32),
        pltpu.VMEM((16, BPW), jnp.float32),
        pltpu.VMEM((BPW,), jnp.float32),
        pltpu.SemaphoreType.DMA,
    ),
    compiler_params=pltpu.CompilerParams(use_tc_tiling_on_sc=False),
  )
  def _sc_gather(w1_hbm, b1_hbm, w2_hbm, b2_hbm, ix_hbm,
                 w1o, b1o, w2o, b2o,
                 idx_v, idxp_v, g_v, b2v, sem):
    wid = lax.axis_index("s") * NC + lax.axis_index("c")
    base = wid * BPW
    pltpu.sync_copy(ix_hbm.at[pl.ds(base, BPW)], idx_v)

    def round16(tab_hbm, out_hbm, row0, scale, stride):
        for p in range(16):
            off = (row0 + p) * stride
            for j in range(BPW // 16):
                v = idx_v[pl.ds(j * 16, 16)]
                idxp_v[p, pl.ds(j * 16, 16)] = v * scale + off
        copies = [pltpu.async_copy(tab_hbm.at[idxp_v.at[p]], g_v.at[p], sem)
                  for p in range(16)]
        for c in copies:
            c.wait()
        pltpu.sync_copy(g_v, out_hbm.at[pl.ds(row0, 16), pl.ds(base, BPW)])

    def w1_round(s, carry):
        round16(w1_hbm, w1o, s * 16, 1, N_REGIONS)
        return carry

    lax.fori_loop(0, 16, w1_round, 0)
    round16(b1_hbm, b1o, 0, 1, N_REGIONS)
    round16(w2_hbm, w2o, 0, 16, 1)

    pltpu.async_copy(b2_hbm.at[idx_v], b2v, sem).wait()
    pltpu.sync_copy(b2v, b2o.at[pl.ds(base, BPW)])

  return _sc_gather


B_R = 512
C_CH = 32


def _tc_body(xt_ref, w1_ref, b1_ref, w2_ref, b2_ref, out_ref):
    def chunk(i, carry):
        a0 = pl.multiple_of(i * C_CH, C_CH)
        acc = jnp.broadcast_to(b2_ref[0:1, :], (C_CH, B_R))
        for d in range(N_INTER):
            hd = jnp.broadcast_to(b1_ref[d:d + 1, :], (C_CH, B_R))
            for c in range(N_EMB):
                hd = hd + xt_ref[pl.ds(a0, C_CH), c, :] * w1_ref[16 * c + d:16 * c + d + 1, :]
            hs = 0.5 * jnp.tanh(0.5 * hd) + 0.5
            acc = acc + hs * w2_ref[d:d + 1, :]
        out_ref[pl.ds(a0, C_CH), :] = acc
        return carry

    lax.fori_loop(0, N_CELLS // C_CH, chunk, 0)


def _tc_compute(xt, w1cd, b1t, w2t, b2row):
    nb = N_REG_B // B_R
    return pl.pallas_call(
        _tc_body,
        grid=(nb,),
        in_specs=[
            pl.BlockSpec((N_CELLS, N_EMB, B_R), lambda i: (0, 0, i)),
            pl.BlockSpec((N_EMB * N_INTER, B_R), lambda i: (0, i)),
            pl.BlockSpec((N_INTER, B_R), lambda i: (0, i)),
            pl.BlockSpec((N_INTER, B_R), lambda i: (0, i)),
            pl.BlockSpec((1, B_R), lambda i: (0, i)),
        ],
        out_specs=pl.BlockSpec((N_CELLS, B_R), lambda i: (0, i)),
        out_shape=jax.ShapeDtypeStruct((N_CELLS, N_REG_B), jnp.float32),
    )(xt, w1cd, b1t, w2t, b2row)


def kernel(cell_region_embedding, region_ix, W1, b1, W2, b2):
    ix = region_ix.astype(jnp.int32)
    w1pf = jnp.transpose(W1, (1, 2, 0)).reshape(N_EMB * N_INTER * N_REGIONS)
    b1pf = b1.T.reshape(N_INTER * N_REGIONS)
    w2f = W2.reshape(N_REGIONS * N_INTER)
    b2f = b2.reshape(N_REGIONS)

    w1cd, b1t, w2t, b2g = _make_sc_gather()(w1pf, b1pf, w2f, b2f, ix)

    xt = jnp.transpose(cell_region_embedding, (0, 2, 1))
    return _tc_compute(xt, w1cd, b1t, w2t, b2g.reshape(1, N_REG_B))


# speedup vs baseline: 2.5100x; 1.8431x over previous
import functools

import jax
import jax.numpy as jnp
from jax import lax
from jax.experimental import pallas as pl
from jax.experimental.pallas import tpu as pltpu
from jax.experimental.pallas import tpu_sc as plsc

N_REGIONS = 100000
N_EMB = 16
N_INTER = 16
N_CELLS = 256
N_REG_B = 4096

NC = 2
NS = 16
NW = NC * NS
BPW = N_REG_B // NW


@functools.cache
def _make_sc_gather():
  mesh = plsc.VectorSubcoreMesh(core_axis_name="c", subcore_axis_name="s")

  @functools.partial(
    pl.kernel,
    mesh=mesh,
    out_type=(
        jax.ShapeDtypeStruct((N_EMB * N_INTER, N_REG_B), jnp.float32),
        jax.ShapeDtypeStruct((N_INTER, N_REG_B), jnp.float32),
        jax.ShapeDtypeStruct((N_INTER, N_REG_B), jnp.float32),
        jax.ShapeDtypeStruct((N_REG_B,), jnp.float32),
    ),
    scratch_types=(
        pltpu.VMEM((BPW,), jnp.int32),
        pltpu.VMEM((16, BPW), jnp.int32),
        pltpu.VMEM((16, BPW), jnp.float32),
        pltpu.VMEM((BPW,), jnp.float32),
        pltpu.SemaphoreType.DMA,
    ),
    compiler_params=pltpu.CompilerParams(use_tc_tiling_on_sc=False),
  )
  def _sc_gather(w1_hbm, b1_hbm, w2_hbm, b2_hbm, ix_hbm,
                 w1o, b1o, w2o, b2o,
                 idx_v, idxp_v, g_v, b2v, sem):
    wid = lax.axis_index("s") * NC + lax.axis_index("c")
    base = wid * BPW
    pltpu.sync_copy(ix_hbm.at[pl.ds(base, BPW)], idx_v)

    def round16(tab_hbm, out_hbm, row0, scale, stride):
        for p in range(16):
            off = (row0 + p) * stride
            for j in range(BPW // 16):
                v = idx_v[pl.ds(j * 16, 16)]
                idxp_v[p, pl.ds(j * 16, 16)] = v * scale + off
        copies = [pltpu.async_copy(tab_hbm.at[idxp_v.at[p]], g_v.at[p], sem)
                  for p in range(16)]
        for c in copies:
            c.wait()
        pltpu.sync_copy(g_v, out_hbm.at[pl.ds(row0, 16), pl.ds(base, BPW)])

    def w1_round(s, carry):
        round16(w1_hbm, w1o, s * 16, 1, N_REGIONS)
        return carry

    lax.fori_loop(0, 16, w1_round, 0)
    round16(b1_hbm, b1o, 0, 1, N_REGIONS)
    round16(w2_hbm, w2o, 0, 16, 1)

    pltpu.async_copy(b2_hbm.at[idx_v], b2v, sem).wait()
    pltpu.sync_copy(b2v, b2o.at[pl.ds(base, BPW)])

  return _sc_gather


B_R = 512
C_CH = 32


def _tc_body(xt_ref, w1_ref, b1_ref, w2_ref, b2_ref, out_ref, xs_ref):
    for c in range(N_EMB):
        xs_ref[c] = xt_ref[:, c, :]

    def chunk(i, carry):
        a0 = pl.multiple_of(i * C_CH, C_CH)
        acc = jnp.broadcast_to(b2_ref[0:1, :], (C_CH, B_R))
        for d in range(N_INTER):
            hd = jnp.broadcast_to(b1_ref[d:d + 1, :], (C_CH, B_R))
            for c in range(N_EMB):
                hd = hd + xs_ref[c, pl.ds(a0, C_CH), :] * w1_ref[16 * c + d:16 * c + d + 1, :]
            hs = 0.5 * jnp.tanh(0.5 * hd) + 0.5
            acc = acc + hs * w2_ref[d:d + 1, :]
        out_ref[pl.ds(a0, C_CH), :] = acc
        return carry

    lax.fori_loop(0, N_CELLS // C_CH, chunk, 0)


def _tc_compute(xt, w1cd, b1t, w2t, b2row):
    nb = N_REG_B // B_R
    return pl.pallas_call(
        _tc_body,
        grid=(nb,),
        in_specs=[
            pl.BlockSpec((N_CELLS, N_EMB, B_R), lambda i: (0, 0, i)),
            pl.BlockSpec((N_EMB * N_INTER, B_R), lambda i: (0, i)),
            pl.BlockSpec((N_INTER, B_R), lambda i: (0, i)),
            pl.BlockSpec((N_INTER, B_R), lambda i: (0, i)),
            pl.BlockSpec((1, B_R), lambda i: (0, i)),
        ],
        out_specs=pl.BlockSpec((N_CELLS, B_R), lambda i: (0, i)),
        out_shape=jax.ShapeDtypeStruct((N_CELLS, N_REG_B), jnp.float32),
        scratch_shapes=[pltpu.VMEM((N_EMB, N_CELLS, B_R), jnp.float32)],
    )(xt, w1cd, b1t, w2t, b2row)


def kernel(cell_region_embedding, region_ix, W1, b1, W2, b2):
    ix = region_ix.astype(jnp.int32)
    w1pf = jnp.transpose(W1, (1, 2, 0)).reshape(N_EMB * N_INTER * N_REGIONS)
    b1pf = b1.T.reshape(N_INTER * N_REGIONS)
    w2f = W2.reshape(N_REGIONS * N_INTER)
    b2f = b2.reshape(N_REGIONS)

    w1cd, b1t, w2t, b2g = _make_sc_gather()(w1pf, b1pf, w2f, b2f, ix)

    xt = jnp.transpose(cell_region_embedding, (0, 2, 1))
    return _tc_compute(xt, w1cd, b1t, w2t, b2g.reshape(1, N_REG_B))


# speedup vs baseline: 2.5729x; 1.0251x over previous
import functools

import jax
import jax.numpy as jnp
from jax import lax
from jax.experimental import pallas as pl
from jax.experimental.pallas import tpu as pltpu
from jax.experimental.pallas import tpu_sc as plsc

N_REGIONS = 100000
N_EMB = 16
N_INTER = 16
N_CELLS = 256
N_REG_B = 4096

NC = 2
NS = 16
NW = NC * NS
BPW = N_REG_B // NW

@functools.cache
def _make_sc_gather():
  mesh = plsc.VectorSubcoreMesh(core_axis_name="c", subcore_axis_name="s")

  @functools.partial(
    pl.kernel,
    mesh=mesh,
    out_type=(
        jax.ShapeDtypeStruct((N_REG_B, N_EMB * N_INTER), jnp.float32),
        jax.ShapeDtypeStruct((N_REG_B, N_INTER), jnp.float32),
        jax.ShapeDtypeStruct((N_REG_B, N_INTER), jnp.float32),
        jax.ShapeDtypeStruct((N_REG_B, 16), jnp.float32),
    ),
    scratch_types=(
        pltpu.VMEM((BPW,), jnp.int32),
        pltpu.VMEM((BPW,), jnp.int32),
        pltpu.VMEM((BPW, N_EMB * N_INTER), jnp.float32),
        pltpu.VMEM((BPW, N_INTER), jnp.float32),
        pltpu.VMEM((BPW, N_INTER), jnp.float32),
        pltpu.VMEM((BPW, 16), jnp.float32),
        pltpu.SemaphoreType.DMA,
    ),
    compiler_params=pltpu.CompilerParams(use_tc_tiling_on_sc=False),
  )
  def _sc_gather(w1_hbm, b1_hbm, w2_hbm, b2_hbm, ix_hbm,
                 w1o, b1o, w2o, b2o,
                 idx_v, idxhi_v, w1v, b1v, w2v, b2v, sem):
    wid = lax.axis_index("s") * NC + lax.axis_index("c")
    base = wid * BPW
    pltpu.sync_copy(ix_hbm.at[pl.ds(base, BPW)], idx_v)

    for j in range(BPW // 16):
        v = idx_v[pl.ds(j * 16, 16)]
        idxhi_v[pl.ds(j * 16, 16)] = lax.shift_right_logical(v, 4)

    pltpu.async_copy(w1_hbm.at[idx_v], w1v, sem).wait()
    pltpu.async_copy(b1_hbm.at[idx_v], b1v, sem).wait()
    pltpu.async_copy(w2_hbm.at[idx_v], w2v, sem).wait()
    pltpu.async_copy(b2_hbm.at[idxhi_v], b2v, sem).wait()

    pltpu.sync_copy(w1v, w1o.at[pl.ds(base, BPW)])
    pltpu.sync_copy(b1v, b1o.at[pl.ds(base, BPW)])
    pltpu.sync_copy(w2v, w2o.at[pl.ds(base, BPW)])
    pltpu.sync_copy(b2v, b2o.at[pl.ds(base, BPW)])

  return _sc_gather


B_R = 512
C_CH = 32


def _tc_body(xt_ref, w1_ref, b1_ref, w2_ref, b2t_ref, lo_ref, out_ref):
    lo = lo_ref[0:1, :]
    b2row = jnp.zeros((1, B_R), jnp.float32)
    for l in range(16):
        b2row = b2row + jnp.where(lo == l, b2t_ref[l:l + 1, :], 0.0)

    def chunk(i, carry):
        a0 = pl.multiple_of(i * C_CH, C_CH)
        acc = jnp.broadcast_to(b2row, (C_CH, B_R))
        for d in range(N_INTER):
            hd = jnp.broadcast_to(b1_ref[d:d + 1, :], (C_CH, B_R))
            for c in range(N_EMB):
                hd = hd + xt_ref[c, pl.ds(a0, C_CH), :] * w1_ref[16 * c + d:16 * c + d + 1, :]
            hs = 0.5 * jnp.tanh(0.5 * hd) + 0.5
            acc = acc + hs * w2_ref[d:d + 1, :]
        out_ref[pl.ds(a0, C_CH), :] = acc
        return carry

    lax.fori_loop(0, N_CELLS // C_CH, chunk, 0)


def _tc_compute(xt, w1cd, b1t, w2t, b2t, lorow):
    nb = N_REG_B // B_R
    return pl.pallas_call(
        _tc_body,
        grid=(nb,),
        in_specs=[
            pl.BlockSpec((N_EMB, N_CELLS, B_R), lambda i: (0, 0, i)),
            pl.BlockSpec((N_EMB * N_INTER, B_R), lambda i: (0, i)),
            pl.BlockSpec((N_INTER, B_R), lambda i: (0, i)),
            pl.BlockSpec((N_INTER, B_R), lambda i: (0, i)),
            pl.BlockSpec((16, B_R), lambda i: (0, i)),
            pl.BlockSpec((1, B_R), lambda i: (0, i)),
        ],
        out_specs=pl.BlockSpec((N_CELLS, B_R), lambda i: (0, i)),
        out_shape=jax.ShapeDtypeStruct((N_CELLS, N_REG_B), jnp.float32),
    )(xt, w1cd, b1t, w2t, b2t, lorow)


def kernel(cell_region_embedding, region_ix, W1, b1, W2, b2):
    ix = region_ix.astype(jnp.int32)
    w1r = W1.reshape(N_REGIONS, N_EMB * N_INTER)
    w2r = W2.reshape(N_REGIONS, N_INTER)

    b2r = b2.reshape(N_REGIONS // 16, 16)
    w1g, b1g, w2g, b2g = _make_sc_gather()(w1r, b1, w2r, b2r, ix)

    xt = jnp.transpose(cell_region_embedding, (2, 0, 1))
    w1cd = jnp.transpose(w1g.reshape(N_REG_B, N_EMB, N_INTER), (1, 2, 0))
    w1cd = w1cd.reshape(N_EMB * N_INTER, N_REG_B)
    b1t = b1g.T
    w2t = w2g.T
    b2t = b2g.T
    lorow = (ix & 15).reshape(1, N_REG_B)

    return _tc_compute(xt, w1cd, b1t, w2t, b2t, lorow)
